# baseline (speedup 1.0000x reference)
import jax
import jax.numpy as jnp
from jax.experimental import pallas as pl
from jax.experimental.pallas import tpu as pltpu


def _copy_body(gidx_ref, qkv_ref, out_ref):
    out_ref[...] = qkv_ref[...]


def kernel(r_idx, qkv):
    n, p3, w3, c = qkv.shape
    topk = r_idx.shape[-1]
    rows = n * p3
    out_rows = rows * topk
    lanes = 128
    sub = (w3 * c) // lanes

    table = qkv.reshape(rows, sub, lanes)
    gidx = (
        r_idx.astype(jnp.int32)
        + (jnp.arange(n, dtype=jnp.int32) * p3)[:, None, None]
    ).reshape(-1)

    grid_spec = pltpu.PrefetchScalarGridSpec(
        num_scalar_prefetch=1,
        grid=(out_rows,),
        in_specs=[
            pl.BlockSpec((1, sub, lanes), lambda j, gidx_ref: (gidx_ref[j], 0, 0)),
        ],
        out_specs=pl.BlockSpec((1, sub, lanes), lambda j, gidx_ref: (j, 0, 0)),
    )
    out = pl.pallas_call(
        _copy_body,
        grid_spec=grid_spec,
        out_shape=jax.ShapeDtypeStruct((out_rows, sub, lanes), qkv.dtype),
    )(gidx, table)
    return out.reshape(n, p3, topk, w3, c)


# speedup vs baseline: 3.3957x; 3.3957x over previous
import jax
import jax.numpy as jnp
from jax.experimental import pallas as pl
from jax.experimental.pallas import tpu as pltpu

_G = 8


def _body(gidx_ref, table_hbm, out_ref, table_vmem, sem):
    j = pl.program_id(0)

    @pl.when(j == 0)
    def _():
        cp = pltpu.make_async_copy(table_hbm, table_vmem, sem)
        cp.start()
        cp.wait()

    for t in range(_G):
        idx = gidx_ref[j * _G + t]
        out_ref[t] = table_vmem[idx]


def kernel(r_idx, qkv):
    n, p3, w3, c = qkv.shape
    topk = r_idx.shape[-1]
    rows = n * p3
    out_rows = rows * topk
    lanes = 128
    sub = (w3 * c) // lanes

    table = qkv.reshape(rows, sub, lanes)
    gidx = (
        r_idx.astype(jnp.int32)
        + (jnp.arange(n, dtype=jnp.int32) * p3)[:, None, None]
    ).reshape(-1)

    grid_spec = pltpu.PrefetchScalarGridSpec(
        num_scalar_prefetch=1,
        grid=(out_rows // _G,),
        in_specs=[
            pl.BlockSpec(memory_space=pl.ANY),
        ],
        out_specs=pl.BlockSpec((_G, sub, lanes), lambda j, gidx_ref: (j, 0, 0)),
        scratch_shapes=[
            pltpu.VMEM((rows, sub, lanes), jnp.float32),
            pltpu.SemaphoreType.DMA,
        ],
    )
    out = pl.pallas_call(
        _body,
        grid_spec=grid_spec,
        out_shape=jax.ShapeDtypeStruct((out_rows, sub, lanes), qkv.dtype),
    )(gidx, table)
    return out.reshape(n, p3, topk, w3, c)


# speedup vs baseline: 3.7686x; 1.1098x over previous
import jax
import jax.numpy as jnp
from jax.experimental import pallas as pl
from jax.experimental.pallas import tpu as pltpu

_G = 16


def _body(gidx_ref, table_hbm, out_ref, table_vmem, sem):
    j = pl.program_id(0)

    @pl.when(j == 0)
    def _():
        cp = pltpu.make_async_copy(table_hbm, table_vmem, sem)
        cp.start()
        cp.wait()

    for t in range(_G):
        idx = gidx_ref[j * _G + t]
        out_ref[t] = table_vmem[idx]


def kernel(r_idx, qkv):
    n, p3, w3, c = qkv.shape
    topk = r_idx.shape[-1]
    rows = n * p3
    out_rows = rows * topk
    lanes = 128
    sub = (w3 * c) // lanes

    table = qkv.reshape(rows, sub, lanes)
    gidx = (
        r_idx.astype(jnp.int32)
        + (jnp.arange(n, dtype=jnp.int32) * p3)[:, None, None]
    ).reshape(-1)

    grid_spec = pltpu.PrefetchScalarGridSpec(
        num_scalar_prefetch=1,
        grid=(out_rows // _G,),
        in_specs=[
            pl.BlockSpec(memory_space=pl.ANY),
        ],
        out_specs=pl.BlockSpec((_G, sub, lanes), lambda j, gidx_ref: (j, 0, 0)),
        scratch_shapes=[
            pltpu.VMEM((rows, sub, lanes), jnp.float32),
            pltpu.SemaphoreType.DMA,
        ],
    )
    out = pl.pallas_call(
        _body,
        grid_spec=grid_spec,
        out_shape=jax.ShapeDtypeStruct((out_rows, sub, lanes), qkv.dtype),
    )(gidx, table)
    return out.reshape(n, p3, topk, w3, c)
